# baseline (speedup 1.0000x reference)
import functools

import jax
import jax.numpy as jnp
from jax import lax
from jax.experimental import pallas as pl
from jax.experimental.pallas import tpu as pltpu
from jax.experimental.pallas import tpu_sc as plsc

B = 16384
NUM_USERS = 1000000
K = 16
SEM_CODEBOOK = 256
SEM_LEVELS = 4
FIELDS = 1 + SEM_LEVELS
INP = FIELDS * K
B4 = B * SEM_LEVELS

NC = 2
NS = 16
NW = NC * NS
U_PER_W = B // NW
S_PER_W = B4 // NW

_sc_mesh = plsc.VectorSubcoreMesh(core_axis_name="c", subcore_axis_name="s")


@functools.partial(
    pl.kernel,
    out_type=(
        jax.ShapeDtypeStruct((B, K), jnp.float32),
        jax.ShapeDtypeStruct((B, K), jnp.float32),
    ),
    mesh=_sc_mesh,
    scratch_types=[
        pltpu.VMEM((U_PER_W,), jnp.int32),
        pltpu.VMEM((U_PER_W,), jnp.int32),
        pltpu.VMEM((U_PER_W, K), jnp.float32),
        pltpu.VMEM((U_PER_W, K), jnp.float32),
        pltpu.SemaphoreType.DMA,
    ],
    compiler_params=pltpu.CompilerParams(use_tc_tiling_on_sc=False),
)
def _sc_gather_user(uidxa_hbm, uidxb_hbm, utaba_hbm, utabb_hbm,
                    uveca_out, uvecb_out,
                    uidxa_v, uidxb_v, urowsa_v, urowsb_v, sem):
    wid = lax.axis_index("s") * NC + lax.axis_index("c")
    ub = wid * U_PER_W
    pltpu.sync_copy(uidxa_hbm.at[pl.ds(ub, U_PER_W)], uidxa_v)
    pltpu.sync_copy(uidxb_hbm.at[pl.ds(ub, U_PER_W)], uidxb_v)
    ca = pltpu.async_copy(utaba_hbm.at[uidxa_v], urowsa_v, sem)
    cb = pltpu.async_copy(utabb_hbm.at[uidxb_v], urowsb_v, sem)
    ca.wait()
    cb.wait()
    pltpu.sync_copy(urowsa_v, uveca_out.at[pl.ds(ub, U_PER_W)])
    pltpu.sync_copy(urowsb_v, uvecb_out.at[pl.ds(ub, U_PER_W)])


@functools.partial(
    pl.kernel,
    out_type=(
        jax.ShapeDtypeStruct((B, K), jnp.float32),
        jax.ShapeDtypeStruct((B4, K), jnp.float32),
        jax.ShapeDtypeStruct((B4, K), jnp.float32),
    ),
    mesh=_sc_mesh,
    scratch_types=[
        pltpu.VMEM((U_PER_W,), jnp.int32),
        pltpu.VMEM((S_PER_W,), jnp.int32),
        pltpu.VMEM((S_PER_W,), jnp.int32),
        pltpu.VMEM((U_PER_W, K), jnp.float32),
        pltpu.VMEM((S_PER_W, K), jnp.float32),
        pltpu.VMEM((S_PER_W, K), jnp.float32),
        pltpu.SemaphoreType.DMA,
    ],
    compiler_params=pltpu.CompilerParams(use_tc_tiling_on_sc=False),
)
def _sc_gather_small(ubidx_hbm, sidx_hbm, sbidx_hbm,
                     ubias_hbm, stab_hbm, sbias_hbm,
                     ubr_out, svec_out, sbr_out,
                     ubidx_v, sidx_v, sbidx_v,
                     ubrows_v, srows_v, sbrows_v, sem):
    wid = lax.axis_index("s") * NC + lax.axis_index("c")
    ub = wid * U_PER_W
    sb = wid * S_PER_W
    pltpu.sync_copy(ubidx_hbm.at[pl.ds(ub, U_PER_W)], ubidx_v)
    pltpu.sync_copy(sidx_hbm.at[pl.ds(sb, S_PER_W)], sidx_v)
    pltpu.sync_copy(sbidx_hbm.at[pl.ds(sb, S_PER_W)], sbidx_v)
    copies = [
        pltpu.async_copy(ubias_hbm.at[ubidx_v], ubrows_v, sem),
        pltpu.async_copy(stab_hbm.at[sidx_v], srows_v, sem),
        pltpu.async_copy(sbias_hbm.at[sbidx_v], sbrows_v, sem),
    ]
    for c in copies:
        c.wait()
    pltpu.sync_copy(ubrows_v, ubr_out.at[pl.ds(ub, U_PER_W)])
    pltpu.sync_copy(srows_v, svec_out.at[pl.ds(sb, S_PER_W)])
    pltpu.sync_copy(sbrows_v, sbr_out.at[pl.ds(sb, S_PER_W)])


R = 4096


def _dense_body(uveca, uvecb, upick, ubr, ulane, svec, sbr, slane, W1, b1, W2, b2, W3, b3, out):
    u = jnp.where(upick[...] == 0, uveca[...], uvecb[...])
    s = svec[...]
    x = jnp.concatenate([u, s], axis=1)
    sum_vec = u + s[:, 0:16] + s[:, 16:32] + s[:, 32:48] + s[:, 48:64]
    sum_sq = jnp.sum(sum_vec * sum_vec, axis=1, keepdims=True)
    sq_sum = jnp.sum(x * x, axis=1, keepdims=True)
    fm2 = 0.5 * (sum_sq - sq_sum)

    iota16 = lax.broadcasted_iota(jnp.int32, (R, K), 1)
    first = jnp.sum(jnp.where(iota16 == ulane[...], ubr[...], 0.0),
                    axis=1, keepdims=True)
    sl = slane[...]
    sbv = sbr[...]
    for l in range(SEM_LEVELS):
        first = first + jnp.sum(
            jnp.where(iota16 == sl[:, l:l + 1], sbv[:, l * K:(l + 1) * K], 0.0),
            axis=1, keepdims=True)

    h = jnp.dot(x, W1[...], preferred_element_type=jnp.float32) + b1[...][None, :]
    h = jnp.maximum(h, 0.0)
    h = jnp.dot(h, W2[...], preferred_element_type=jnp.float32) + b2[...][None, :]
    h = jnp.maximum(h, 0.0)
    deep = jnp.dot(h, W3[...], preferred_element_type=jnp.float32) + b3[...][None, :]
    logits = first + fm2 + deep
    out[...] = (1.0 / (1.0 + jnp.exp(-logits)))[:, 0]


_dense = pl.pallas_call(
    _dense_body,
    grid=(B // R,),
    in_specs=[
        pl.BlockSpec((R, K), lambda i: (i, 0)),
        pl.BlockSpec((R, K), lambda i: (i, 0)),
        pl.BlockSpec((R, 1), lambda i: (i, 0)),
        pl.BlockSpec((R, K), lambda i: (i, 0)),
        pl.BlockSpec((R, 1), lambda i: (i, 0)),
        pl.BlockSpec((R, SEM_LEVELS * K), lambda i: (i, 0)),
        pl.BlockSpec((R, SEM_LEVELS * K), lambda i: (i, 0)),
        pl.BlockSpec((R, SEM_LEVELS), lambda i: (i, 0)),
        pl.BlockSpec((INP, 128), lambda i: (0, 0)),
        pl.BlockSpec((128,), lambda i: (0,)),
        pl.BlockSpec((128, 64), lambda i: (0, 0)),
        pl.BlockSpec((64,), lambda i: (0,)),
        pl.BlockSpec((64, 1), lambda i: (0, 0)),
        pl.BlockSpec((1,), lambda i: (0,)),
    ],
    out_specs=pl.BlockSpec((R,), lambda i: (i,)),
    out_shape=jax.ShapeDtypeStruct((B,), jnp.float32),
)


def kernel(user, sem_codes, user_table, user_bias, sem_tables, sem_biases,
           W1, b1, W2, b2, W3, b3):
    ui = user.astype(jnp.int32)
    uidx = ui
    ubidx = ui >> 4
    ulane = (ui & 15).reshape(B, 1)
    codes = jnp.clip(sem_codes, 0, SEM_CODEBOOK - 1).astype(jnp.int32)
    sflat = (codes + (jnp.arange(SEM_LEVELS, dtype=jnp.int32) * SEM_CODEBOOK)[None, :]).reshape(-1)
    sidx = sflat
    sbidx = sflat >> 4
    slane = (codes & 15)
    stab = sem_tables.reshape(SEM_LEVELS * SEM_CODEBOOK, K)
    ubias16 = user_bias.reshape(NUM_USERS // K, K)
    sbias16 = sem_biases.reshape(SEM_LEVELS * SEM_CODEBOOK // K, K)

    half = NUM_USERS // 2
    uidxa = jnp.minimum(ui, half - 1)
    uidxb = jnp.minimum(jnp.maximum(ui - half, 0), half - 1)
    upick = (ui >= half).astype(jnp.int32).reshape(B, 1)
    ubr, svec, sbr = _sc_gather_small(ubidx, sidx, sbidx, ubias16, stab, sbias16)
    uveca, uvecb = _sc_gather_user(uidxa, uidxb,
                                   user_table[:half], user_table[half:])
    return _dense(
        uveca,
        uvecb,
        upick,
        ubr,
        ulane,
        svec.reshape(B, SEM_LEVELS * K),
        sbr.reshape(B, SEM_LEVELS * K),
        slane,
        W1, b1, W2, b2, W3, b3,
    )


# speedup vs baseline: 1.4399x; 1.4399x over previous
import functools

import jax
import jax.numpy as jnp
from jax import lax
from jax.experimental import pallas as pl
from jax.experimental.pallas import tpu as pltpu
from jax.experimental.pallas import tpu_sc as plsc

B = 16384
NUM_USERS = 1000000
K = 16
SEM_CODEBOOK = 256
SEM_LEVELS = 4
FIELDS = 1 + SEM_LEVELS
INP = FIELDS * K
B4 = B * SEM_LEVELS

NC = 2
NS = 16
NW = NC * NS
U_PER_W = B // NW
S_PER_W = B4 // NW

_sc_mesh = plsc.VectorSubcoreMesh(core_axis_name="c", subcore_axis_name="s")


@functools.partial(
    pl.kernel,
    out_type=jax.ShapeDtypeStruct((B, K), jnp.float32),
    mesh=_sc_mesh,
    scratch_types=[
        pltpu.VMEM((U_PER_W,), jnp.int32),
        pltpu.VMEM((U_PER_W, K), jnp.float32),
        pltpu.SemaphoreType.DMA,
    ],
    compiler_params=pltpu.CompilerParams(use_tc_tiling_on_sc=False),
)
def _sc_gather_user(uidx_hbm, utab_hbm, uvec_out, uidx_v, urows_v, sem):
    wid = lax.axis_index("s") * NC + lax.axis_index("c")
    ub = wid * U_PER_W
    pltpu.sync_copy(uidx_hbm.at[pl.ds(ub, U_PER_W)], uidx_v)
    pltpu.async_copy(utab_hbm.at[uidx_v], urows_v, sem).wait()
    pltpu.sync_copy(urows_v, uvec_out.at[pl.ds(ub, U_PER_W)])


@functools.partial(
    pl.kernel,
    out_type=(
        jax.ShapeDtypeStruct((B, K), jnp.float32),
        jax.ShapeDtypeStruct((B4, K), jnp.float32),
        jax.ShapeDtypeStruct((B4, K), jnp.float32),
    ),
    mesh=_sc_mesh,
    scratch_types=[
        pltpu.VMEM((U_PER_W,), jnp.int32),
        pltpu.VMEM((S_PER_W,), jnp.int32),
        pltpu.VMEM((S_PER_W,), jnp.int32),
        pltpu.VMEM((U_PER_W, K), jnp.float32),
        pltpu.VMEM((S_PER_W, K), jnp.float32),
        pltpu.VMEM((S_PER_W, K), jnp.float32),
        pltpu.SemaphoreType.DMA,
    ],
    compiler_params=pltpu.CompilerParams(use_tc_tiling_on_sc=False),
)
def _sc_gather_small(ubidx_hbm, sidx_hbm, sbidx_hbm,
                     ubias_hbm, stab_hbm, sbias_hbm,
                     ubr_out, svec_out, sbr_out,
                     ubidx_v, sidx_v, sbidx_v,
                     ubrows_v, srows_v, sbrows_v, sem):
    wid = lax.axis_index("s") * NC + lax.axis_index("c")
    ub = wid * U_PER_W
    sb = wid * S_PER_W
    pltpu.sync_copy(ubidx_hbm.at[pl.ds(ub, U_PER_W)], ubidx_v)
    pltpu.sync_copy(sidx_hbm.at[pl.ds(sb, S_PER_W)], sidx_v)
    pltpu.sync_copy(sbidx_hbm.at[pl.ds(sb, S_PER_W)], sbidx_v)
    copies = [
        pltpu.async_copy(ubias_hbm.at[ubidx_v], ubrows_v, sem),
        pltpu.async_copy(stab_hbm.at[sidx_v], srows_v, sem),
        pltpu.async_copy(sbias_hbm.at[sbidx_v], sbrows_v, sem),
    ]
    for c in copies:
        c.wait()
    pltpu.sync_copy(ubrows_v, ubr_out.at[pl.ds(ub, U_PER_W)])
    pltpu.sync_copy(srows_v, svec_out.at[pl.ds(sb, S_PER_W)])
    pltpu.sync_copy(sbrows_v, sbr_out.at[pl.ds(sb, S_PER_W)])


R = 4096


def _dense_body(uvec, ubr, ulane, svec, sbr, slane, W1, b1, W2, b2, W3, b3, out):
    u = uvec[...]
    s = svec[...]
    x = jnp.concatenate([u, s], axis=1)
    sum_vec = u + s[:, 0:16] + s[:, 16:32] + s[:, 32:48] + s[:, 48:64]
    sum_sq = jnp.sum(sum_vec * sum_vec, axis=1, keepdims=True)
    sq_sum = jnp.sum(x * x, axis=1, keepdims=True)
    fm2 = 0.5 * (sum_sq - sq_sum)

    iota16 = lax.broadcasted_iota(jnp.int32, (R, K), 1)
    first = jnp.sum(jnp.where(iota16 == ulane[...], ubr[...], 0.0),
                    axis=1, keepdims=True)
    sl = slane[...]
    sbv = sbr[...]
    for l in range(SEM_LEVELS):
        first = first + jnp.sum(
            jnp.where(iota16 == sl[:, l:l + 1], sbv[:, l * K:(l + 1) * K], 0.0),
            axis=1, keepdims=True)

    h = jnp.dot(x, W1[...], preferred_element_type=jnp.float32) + b1[...][None, :]
    h = jnp.maximum(h, 0.0)
    h = jnp.dot(h, W2[...], preferred_element_type=jnp.float32) + b2[...][None, :]
    h = jnp.maximum(h, 0.0)
    deep = jnp.dot(h, W3[...], preferred_element_type=jnp.float32) + b3[...][None, :]
    logits = first + fm2 + deep
    out[...] = (1.0 / (1.0 + jnp.exp(-logits)))[:, 0]


_dense = pl.pallas_call(
    _dense_body,
    grid=(B // R,),
    in_specs=[
        pl.BlockSpec((R, K), lambda i: (i, 0)),
        pl.BlockSpec((R, K), lambda i: (i, 0)),
        pl.BlockSpec((R, 1), lambda i: (i, 0)),
        pl.BlockSpec((R, SEM_LEVELS * K), lambda i: (i, 0)),
        pl.BlockSpec((R, SEM_LEVELS * K), lambda i: (i, 0)),
        pl.BlockSpec((R, SEM_LEVELS), lambda i: (i, 0)),
        pl.BlockSpec((INP, 128), lambda i: (0, 0)),
        pl.BlockSpec((128,), lambda i: (0,)),
        pl.BlockSpec((128, 64), lambda i: (0, 0)),
        pl.BlockSpec((64,), lambda i: (0,)),
        pl.BlockSpec((64, 1), lambda i: (0, 0)),
        pl.BlockSpec((1,), lambda i: (0,)),
    ],
    out_specs=pl.BlockSpec((R,), lambda i: (i,)),
    out_shape=jax.ShapeDtypeStruct((B,), jnp.float32),
)


def kernel(user, sem_codes, user_table, user_bias, sem_tables, sem_biases,
           W1, b1, W2, b2, W3, b3):
    ui = user.astype(jnp.int32)
    uidx = ui
    ubidx = ui >> 4
    ulane = (ui & 15).reshape(B, 1)
    codes = jnp.clip(sem_codes, 0, SEM_CODEBOOK - 1).astype(jnp.int32)
    sflat = (codes + (jnp.arange(SEM_LEVELS, dtype=jnp.int32) * SEM_CODEBOOK)[None, :]).reshape(-1)
    sidx = sflat
    sbidx = sflat >> 4
    slane = (codes & 15)
    stab = sem_tables.reshape(SEM_LEVELS * SEM_CODEBOOK, K)
    ubias16 = user_bias.reshape(NUM_USERS // K, K)
    sbias16 = sem_biases.reshape(SEM_LEVELS * SEM_CODEBOOK // K, K)

    ubr, svec, sbr = _sc_gather_small(ubidx, sidx, sbidx, ubias16, stab, sbias16)
    uvec = _sc_gather_user(uidx, user_table)
    return _dense(
        uvec,
        ubr,
        ulane,
        svec.reshape(B, SEM_LEVELS * K),
        sbr.reshape(B, SEM_LEVELS * K),
        slane,
        W1, b1, W2, b2, W3, b3,
    )
